# baseline (speedup 1.0000x reference)
import functools

import jax
import jax.numpy as jnp
from jax.experimental import pallas as pl
from jax.experimental.pallas import tpu as pltpu

_K = 1024
_LAMDA = 0.5
_T = 5.0


def _body(x_ref, xt_ref, tgt_ref, out_ref, keys_ref, *, rows, v):
    inv_t = jnp.float32(1.0 / _T)
    xs = x_ref[...] * inv_t
    xt = xt_ref[...] * inv_t

    bits = jax.lax.bitcast_convert_type(xt, jnp.uint32)
    keys_ref[...] = jnp.where(
        bits >= jnp.uint32(0x80000000), ~bits, bits | jnp.uint32(0x80000000)
    )

    idx = jax.lax.broadcasted_iota(jnp.int32, (rows, v), 1)

    max_s = jnp.max(xs, axis=1, keepdims=True)
    sum_s = jnp.sum(jnp.exp(xs - max_s), axis=1)
    tgt = tgt_ref[...]
    xs_tgt = jnp.sum(jnp.where(idx == tgt, xs, 0.0), axis=1)
    max_t = jnp.max(xt, axis=1, keepdims=True)

    kc = jnp.int32(_K)

    def vbody(_, carry):
        lo, hi, cnt_lo = carry
        gap = hi - lo
        mid = lo + (gap >> 1) + (gap & jnp.uint32(1))
        cnt = jnp.count_nonzero(keys_ref[...] >= mid, axis=1,
                                keepdims=True).astype(jnp.int32)
        pred = cnt >= kc
        return (jnp.where(pred, mid, lo), jnp.where(pred, hi, mid - 1),
                jnp.where(pred, cnt, cnt_lo))

    tau, _, cnt_ge = jax.lax.fori_loop(
        0, 32, vbody,
        (jnp.zeros((rows, 1), jnp.uint32),
         jnp.full((rows, 1), 0xFFFFFFFF, jnp.uint32),
         jnp.full((rows, 1), v, jnp.int32)),
    )

    def _no_tie(_):
        return jnp.full((rows, 1), v - 1, jnp.int32)

    def _tie(_):
        eq = keys_ref[...] == tau
        cnt_eq = jnp.sum(eq.astype(jnp.int32), axis=1, keepdims=True)
        r = kc - (cnt_ge - cnt_eq)

        def ibody(_, carry):
            lo, hi = carry
            mid = (lo + hi) >> 1
            cnt = jnp.sum((eq & (idx <= mid)).astype(jnp.int32), axis=1,
                          keepdims=True)
            pred = cnt >= r
            return jnp.where(pred, lo, mid + 1), jnp.where(pred, mid, hi)

        istar, _ = jax.lax.fori_loop(
            0, 17, ibody,
            (jnp.zeros((rows, 1), jnp.int32),
             jnp.full((rows, 1), v - 1, jnp.int32)),
        )
        return istar

    istar = jax.lax.cond(jnp.all(cnt_ge == kc), _no_tie, _tie, 0)

    keys = keys_ref[...]
    mask = (keys > tau) | ((keys == tau) & (idx <= istar))

    xs_m = jnp.where(mask, xs, -jnp.inf)
    max_g = jnp.max(xs_m, axis=1, keepdims=True)
    e_s = jnp.where(mask, jnp.exp(xs - max_g), 0.0)
    a = jnp.sum(e_s, axis=1)
    b = jnp.sum(e_s * (xs - xt), axis=1)
    c = jnp.sum(jnp.where(mask, jnp.exp(xt - max_t), 0.0), axis=1)

    kl = b / a - (max_g[:, 0] + jnp.log(a)) + (max_t[:, 0] + jnp.log(c))
    ce = -(xs_tgt - max_s[:, 0] - jnp.log(sum_s))
    total = jnp.sum(ce + jnp.float32(_LAMDA * _T * _T) * kl)

    @pl.when(pl.program_id(0) == 0)
    def _init():
        out_ref[...] = jnp.zeros((1, 1), jnp.float32)

    out_ref[...] += total.reshape(1, 1)


def kernel(x, target, x_teacher):
    bsz, v = x.shape
    rows = 16
    grid = bsz // rows
    tgt2 = target.reshape(bsz, 1).astype(jnp.int32)
    out = pl.pallas_call(
        functools.partial(_body, rows=rows, v=v),
        grid=(grid,),
        in_specs=[
            pl.BlockSpec((rows, v), lambda i: (i, 0)),
            pl.BlockSpec((rows, v), lambda i: (i, 0)),
            pl.BlockSpec((rows, 1), lambda i: (i, 0)),
        ],
        out_specs=pl.BlockSpec((1, 1), lambda i: (0, 0)),
        out_shape=jax.ShapeDtypeStruct((1, 1), jnp.float32),
        scratch_shapes=[pltpu.VMEM((rows, v), jnp.uint32)],
        compiler_params=pltpu.CompilerParams(
            dimension_semantics=("arbitrary",)),
    )(x, x_teacher, tgt2)
    return out[0, 0] / bsz


# speedup vs baseline: 1.0629x; 1.0629x over previous
import functools

import jax
import jax.numpy as jnp
from jax.experimental import pallas as pl
from jax.experimental.pallas import tpu as pltpu

_K = 1024
_LAMDA = 0.5
_T = 5.0


def _key_to_f32(key):
    bits = jnp.where(key >= jnp.uint32(0x80000000),
                     key ^ jnp.uint32(0x80000000), ~key)
    return jax.lax.bitcast_convert_type(bits, jnp.float32)


def _body(x_ref, xt_ref, tgt_ref, out_ref, *, rows, v):
    scale = jnp.float32(1.0 / _T)
    kc = jnp.int32(_K)
    idx = jax.lax.broadcasted_iota(jnp.int32, (rows, v), 1)

    maxx = jnp.max(x_ref[...], axis=1, keepdims=True)
    maxt = jnp.max(xt_ref[...], axis=1, keepdims=True)
    sum_s = jnp.sum(jnp.exp((x_ref[...] - maxx) * scale), axis=1)
    tgt = tgt_ref[...]
    x_tgt = jnp.sum(jnp.where(idx == tgt, x_ref[...], 0.0), axis=1)

    def vbody(_, carry):
        lo, hi, cnt_lo = carry
        gap = hi - lo
        mid = lo + (gap >> 1) + (gap & jnp.uint32(1))
        fmid = _key_to_f32(mid)
        cnt = jnp.sum((xt_ref[...] >= fmid).astype(jnp.int32), axis=1,
                      keepdims=True)
        pred = cnt >= kc
        return (jnp.where(pred, mid, lo), jnp.where(pred, hi, mid - 1),
                jnp.where(pred, cnt, cnt_lo))

    tau, _, cnt_ge = jax.lax.fori_loop(
        0, 32, vbody,
        (jnp.zeros((rows, 1), jnp.uint32),
         jnp.full((rows, 1), 0xFFFFFFFF, jnp.uint32),
         jnp.full((rows, 1), v, jnp.int32)),
    )
    ftau = _key_to_f32(tau)

    def _no_tie(_):
        return jnp.full((rows, 1), v - 1, jnp.int32)

    def _tie(_):
        eq = xt_ref[...] == ftau
        cnt_eq = jnp.sum(eq.astype(jnp.int32), axis=1, keepdims=True)
        r = kc - (cnt_ge - cnt_eq)

        def ibody(_, carry):
            lo, hi = carry
            mid = (lo + hi) >> 1
            cnt = jnp.sum((eq & (idx <= mid)).astype(jnp.int32), axis=1,
                          keepdims=True)
            pred = cnt >= r
            return jnp.where(pred, lo, mid + 1), jnp.where(pred, mid, hi)

        istar, _ = jax.lax.fori_loop(
            0, 17, ibody,
            (jnp.zeros((rows, 1), jnp.int32),
             jnp.full((rows, 1), v - 1, jnp.int32)),
        )
        return istar

    istar = jax.lax.cond(jnp.all(cnt_ge == kc), _no_tie, _tie, 0)

    mask = (xt_ref[...] > ftau) | ((xt_ref[...] == ftau) & (idx <= istar))

    e_s = jnp.where(mask, jnp.exp((x_ref[...] - maxx) * scale), 0.0)
    a = jnp.sum(e_s, axis=1)
    b = jnp.sum(e_s * (x_ref[...] - xt_ref[...]), axis=1) * scale
    c = jnp.sum(jnp.where(mask, jnp.exp((xt_ref[...] - maxt) * scale), 0.0),
                axis=1)

    kl = b / a - jnp.log(a) + jnp.log(c) + (maxt[:, 0] - maxx[:, 0]) * scale
    ce = -((x_tgt - maxx[:, 0]) * scale - jnp.log(sum_s))
    total = jnp.sum(ce + jnp.float32(_LAMDA * _T * _T) * kl)

    @pl.when(pl.program_id(0) == 0)
    def _init():
        out_ref[...] = jnp.zeros((1, 1), jnp.float32)

    out_ref[...] += total.reshape(1, 1)


def kernel(x, target, x_teacher):
    bsz, v = x.shape
    rows = 16
    grid = bsz // rows
    tgt2 = target.reshape(bsz, 1).astype(jnp.int32)
    out = pl.pallas_call(
        functools.partial(_body, rows=rows, v=v),
        grid=(grid,),
        in_specs=[
            pl.BlockSpec((rows, v), lambda i: (i, 0)),
            pl.BlockSpec((rows, v), lambda i: (i, 0)),
            pl.BlockSpec((rows, 1), lambda i: (i, 0)),
        ],
        out_specs=pl.BlockSpec((1, 1), lambda i: (0, 0)),
        out_shape=jax.ShapeDtypeStruct((1, 1), jnp.float32),
        compiler_params=pltpu.CompilerParams(
            dimension_semantics=("arbitrary",)),
    )(x, x_teacher, tgt2)
    return out[0, 0] / bsz
